# Initial kernel scaffold; baseline (speedup 1.0000x reference)
import jax
import jax.numpy as jnp
from jax.experimental import pallas as pl


def kernel(x, edge_index, mask, batch, W1, b1, W2, b2, Wl, bl):
    raise NotImplementedError("write your pallas kernel here")


# speedup vs baseline: 1.3140x; 1.3140x over previous
import functools

import jax
import jax.numpy as jnp
from jax.experimental import pallas as pl
from jax.experimental.pallas import tpu as pltpu

N_NODES = 10000
HIDDEN = 256
NUM_GRAPHS = 64
ROW_BLOCK = 1000


def _mm_kernel(x_ref, w_ref, o_ref):
    o_ref[...] = jnp.dot(x_ref[...], w_ref[...],
                         preferred_element_type=jnp.float32)


def _elu_mm_kernel(x_ref, b_ref, w_ref, o_ref):
    h = x_ref[...] + b_ref[...]
    h = jnp.where(h > 0, h, jnp.exp(h) - 1.0)
    o_ref[...] = jnp.dot(h, w_ref[...], preferred_element_type=jnp.float32)


def _matmul(x, w):
    n = x.shape[0]
    return pl.pallas_call(
        _mm_kernel,
        grid=(n // ROW_BLOCK,),
        in_specs=[
            pl.BlockSpec((ROW_BLOCK, x.shape[1]), lambda i: (i, 0)),
            pl.BlockSpec((w.shape[0], w.shape[1]), lambda i: (0, 0)),
        ],
        out_specs=pl.BlockSpec((ROW_BLOCK, w.shape[1]), lambda i: (i, 0)),
        out_shape=jax.ShapeDtypeStruct((n, w.shape[1]), jnp.float32),
    )(x, w)


def _elu_matmul(x, b, w):
    n = x.shape[0]
    return pl.pallas_call(
        _elu_mm_kernel,
        grid=(n // ROW_BLOCK,),
        in_specs=[
            pl.BlockSpec((ROW_BLOCK, x.shape[1]), lambda i: (i, 0)),
            pl.BlockSpec((1, x.shape[1]), lambda i: (0, 0)),
            pl.BlockSpec((w.shape[0], w.shape[1]), lambda i: (0, 0)),
        ],
        out_specs=pl.BlockSpec((ROW_BLOCK, w.shape[1]), lambda i: (i, 0)),
        out_shape=jax.ShapeDtypeStruct((n, w.shape[1]), jnp.float32),
    )(x, b[None, :], w)


def _pool_kernel(x_ref, b_ref, batch_ref, wl_ref, bl_ref, o_ref, pool_ref):
    h = x_ref[...] + b_ref[...]
    h = jnp.where(h > 0, h, jnp.exp(h) - 1.0)
    seg = batch_ref[...]
    neg_inf = jnp.float32(-jnp.inf)

    def body(g, _):
        m = (seg == g)
        vals = jnp.where(m, h, neg_inf)
        pool_ref[pl.ds(g, 1), :] = jnp.max(vals, axis=0)[None, :]
        return 0

    jax.lax.fori_loop(0, NUM_GRAPHS, body, 0)
    pooled = pool_ref[...]
    logits = jnp.dot(pooled, wl_ref[...],
                     preferred_element_type=jnp.float32) + bl_ref[...]
    mx = jnp.max(logits, axis=1, keepdims=True)
    z = logits - mx
    lse = jnp.log(jnp.sum(jnp.exp(z), axis=1, keepdims=True))
    o_ref[...] = z - lse


def _pool_classify(x, b2, batch, wl, bl):
    mask_n = x.shape[0]
    ncls = wl.shape[1]
    return pl.pallas_call(
        _pool_kernel,
        out_shape=jax.ShapeDtypeStruct((NUM_GRAPHS, ncls), jnp.float32),
        scratch_shapes=[pltpu.VMEM((NUM_GRAPHS, HIDDEN), jnp.float32)],
    )(x, b2[None, :], batch[:, None], wl, bl[None, :])


@jax.jit
def kernel(x, edge_index, mask, batch, W1, b1, W2, b2, Wl, bl):
    row = edge_index[0]
    col = edge_index[1]
    n = x.shape[0]
    deg = jnp.ones((n,), jnp.float32).at[col].add(1.0)
    dinv = jax.lax.rsqrt(deg)
    norm_e = dinv[row] * dinv[col]
    self_w = dinv * dinv

    def aggregate(xw):
        msg = xw[row] * norm_e[:, None]
        out = (xw * self_w[:, None]).at[col].add(msg)
        return out

    xw1 = _matmul(x, W1)
    agg1 = aggregate(xw1)
    xw2 = _elu_matmul(agg1, b1, W2)
    agg2 = aggregate(xw2)
    X = agg2[mask]
    return _pool_classify(X, b2, batch, Wl, bl)
